# baseline (speedup 1.0000x reference)
import functools

import jax
import jax.numpy as jnp
from jax import lax
from jax.experimental import pallas as pl
from jax.experimental.pallas import tpu as pltpu
from jax.experimental.pallas import tpu_sc as plsc

EMBED_DIM = 128
BATCH = 4
SEQ = 2048
SIDE_TOKENS = BATCH * SEQ
NC, NS, L = 2, 16, 16
NW = NC * NS
CHUNK = 64
PER_SIDE = SIDE_TOKENS // NW
CHUNKS_PER_SIDE = PER_SIDE // CHUNK
N_CHUNKS = 2 * CHUNKS_PER_SIDE
W_PER_ROW = SEQ // PER_SIDE
ROWS_UNROLL = 2


@functools.partial(
    pl.kernel,
    out_type=(
        jax.ShapeDtypeStruct((SIDE_TOKENS, EMBED_DIM), jnp.float32),
        jax.ShapeDtypeStruct((SIDE_TOKENS, EMBED_DIM), jnp.float32),
    ),
    mesh=plsc.VectorSubcoreMesh(core_axis_name="c", subcore_axis_name="s"),
    scratch_types=[
        pltpu.VMEM((N_CHUNKS * CHUNK,), jnp.int32),
        pltpu.VMEM((N_CHUNKS * CHUNK, EMBED_DIM), jnp.float32),
        pltpu.VMEM((2, L), jnp.float32),
        pltpu.SemaphoreType.DMA,
        pltpu.SemaphoreType.DMA,
        pltpu.SemaphoreType.DMA,
        pltpu.SemaphoreType.DMA,
        pltpu.SemaphoreType.DMA,
        pltpu.SemaphoreType.DMA,
        pltpu.SemaphoreType.DMA,
        pltpu.SemaphoreType.DMA,
        pltpu.SemaphoreType.DMA,
    ],
)
def _embed_kernel(enc_ids, dec_ids, scales, table,
                  enc_out, dec_out,
                  idx_v, rows_v, scale_v, g0, g1, g2, g3, g4, g5, g6, g7, osem):
    wid = lax.axis_index("s") * NC + lax.axis_index("c")
    b = wid // W_PER_ROW
    col = (wid % W_PER_ROW) * PER_SIDE

    gsems = [g0, g1, g2, g3, g4, g5, g6, g7]
    gathers = []
    for side, ids in enumerate((enc_ids, dec_ids)):
        off = side * PER_SIDE
        pltpu.sync_copy(ids.at[b, pl.ds(col, PER_SIDE)],
                        idx_v.at[pl.ds(off, PER_SIDE)])
        for jj in range(CHUNKS_PER_SIDE):
            j = side * CHUNKS_PER_SIDE + jj
            gathers.append(
                pltpu.async_copy(table.at[idx_v.at[pl.ds(j * CHUNK, CHUNK)]],
                                 rows_v.at[pl.ds(j * CHUNK, CHUNK)], gsems[j])
            )
    pltpu.sync_copy(scales, scale_v)

    writes = []
    for j in range(N_CHUNKS):
        gathers[j].wait()
        side = j // CHUNKS_PER_SIDE
        s = scale_v[side, :]
        base = j * CHUNK

        def scale_rows(i, carry, base=base, s=s):
            row = base + i * ROWS_UNROLL
            for r in range(ROWS_UNROLL):
                for cs in range(EMBED_DIM // L):
                    sl = pl.ds(cs * L, L)
                    rows_v[row + r, sl] = rows_v[row + r, sl] * s
            return carry

        lax.fori_loop(0, CHUNK // ROWS_UNROLL, scale_rows, 0)

        dst = enc_out if side == 0 else dec_out
        off = wid * PER_SIDE + (j % CHUNKS_PER_SIDE) * CHUNK
        writes.append(
            pltpu.async_copy(rows_v.at[pl.ds(base, CHUNK)],
                             dst.at[pl.ds(off, CHUNK)], osem)
        )
    for w in writes:
        w.wait()


def kernel(input_ids, encoder_embed_scale, decoder_input_ids, decoder_embed_scale, table):
    batch, enc_len = input_ids.shape
    dec_len = decoder_input_ids.shape[1]
    scales = jnp.broadcast_to(
        jnp.stack([encoder_embed_scale, decoder_embed_scale]).astype(jnp.float32)[:, None],
        (2, L),
    )
    enc, dec = _embed_kernel(input_ids.astype(jnp.int32),
                             decoder_input_ids.astype(jnp.int32),
                             scales, table)
    return (enc.reshape(batch, enc_len, EMBED_DIM),
            dec.reshape(batch, dec_len, EMBED_DIM))


# speedup vs baseline: 1.0945x; 1.0945x over previous
import functools

import jax
import jax.numpy as jnp
from jax import lax
from jax.experimental import pallas as pl
from jax.experimental.pallas import tpu as pltpu
from jax.experimental.pallas import tpu_sc as plsc

EMBED_DIM = 128
BATCH = 4
SEQ = 2048
SIDE_TOKENS = BATCH * SEQ
NC, NS, L = 2, 16, 16
NW = NC * NS
CHUNK = 128
PER_SIDE = SIDE_TOKENS // NW
CHUNKS_PER_SIDE = PER_SIDE // CHUNK
N_CHUNKS = 2 * CHUNKS_PER_SIDE
W_PER_ROW = SEQ // PER_SIDE
ROWS_UNROLL = 2


@functools.partial(
    pl.kernel,
    out_type=(
        jax.ShapeDtypeStruct((SIDE_TOKENS, EMBED_DIM), jnp.float32),
        jax.ShapeDtypeStruct((SIDE_TOKENS, EMBED_DIM), jnp.float32),
    ),
    mesh=plsc.VectorSubcoreMesh(core_axis_name="c", subcore_axis_name="s"),
    scratch_types=[
        pltpu.VMEM((N_CHUNKS * CHUNK,), jnp.int32),
        pltpu.VMEM((N_CHUNKS * CHUNK, EMBED_DIM), jnp.float32),
        pltpu.VMEM((2, L), jnp.float32),
        pltpu.SemaphoreType.DMA,
        pltpu.SemaphoreType.DMA,
        pltpu.SemaphoreType.DMA,
        pltpu.SemaphoreType.DMA,
        pltpu.SemaphoreType.DMA,
    ],
)
def _embed_kernel(enc_ids, dec_ids, scales, table,
                  enc_out, dec_out,
                  idx_v, rows_v, scale_v, g0, g1, g2, g3, osem):
    wid = lax.axis_index("s") * NC + lax.axis_index("c")
    b = wid // W_PER_ROW
    col = (wid % W_PER_ROW) * PER_SIDE

    gsems = [g0, g1, g2, g3]
    gathers = []
    for side, ids in enumerate((enc_ids, dec_ids)):
        off = side * PER_SIDE
        pltpu.sync_copy(ids.at[b, pl.ds(col, PER_SIDE)],
                        idx_v.at[pl.ds(off, PER_SIDE)])
        for jj in range(CHUNKS_PER_SIDE):
            j = side * CHUNKS_PER_SIDE + jj
            gathers.append(
                pltpu.async_copy(table.at[idx_v.at[pl.ds(j * CHUNK, CHUNK)]],
                                 rows_v.at[pl.ds(j * CHUNK, CHUNK)], gsems[j])
            )
    pltpu.sync_copy(scales, scale_v)

    writes = []
    for j in range(N_CHUNKS):
        gathers[j].wait()
        side = j // CHUNKS_PER_SIDE
        s = scale_v[side, :]
        base = j * CHUNK

        def scale_rows(i, carry, base=base, s=s):
            row = base + i * ROWS_UNROLL
            for r in range(ROWS_UNROLL):
                for cs in range(EMBED_DIM // L):
                    sl = pl.ds(cs * L, L)
                    rows_v[row + r, sl] = rows_v[row + r, sl] * s
            return carry

        lax.fori_loop(0, CHUNK // ROWS_UNROLL, scale_rows, 0)

        if j % CHUNKS_PER_SIDE == CHUNKS_PER_SIDE - 1:
            dst = enc_out if side == 0 else dec_out
            writes.append(
                pltpu.async_copy(rows_v.at[pl.ds(side * PER_SIDE, PER_SIDE)],
                                 dst.at[pl.ds(wid * PER_SIDE, PER_SIDE)], osem)
            )
    for w in writes:
        w.wait()


def kernel(input_ids, encoder_embed_scale, decoder_input_ids, decoder_embed_scale, table):
    batch, enc_len = input_ids.shape
    dec_len = decoder_input_ids.shape[1]
    scales = jnp.broadcast_to(
        jnp.stack([encoder_embed_scale, decoder_embed_scale]).astype(jnp.float32)[:, None],
        (2, L),
    )
    enc, dec = _embed_kernel(input_ids.astype(jnp.int32),
                             decoder_input_ids.astype(jnp.int32),
                             scales, table)
    return (enc.reshape(batch, enc_len, EMBED_DIM),
            dec.reshape(batch, dec_len, EMBED_DIM))


# speedup vs baseline: 1.1144x; 1.0182x over previous
import functools

import jax
import jax.numpy as jnp
from jax import lax
from jax.experimental import pallas as pl
from jax.experimental.pallas import tpu as pltpu
from jax.experimental.pallas import tpu_sc as plsc

EMBED_DIM = 128
BATCH = 4
SEQ = 2048
SIDE_TOKENS = BATCH * SEQ
NC, NS, L = 2, 16, 16
NW = NC * NS
CHUNK = 128
PER_SIDE = SIDE_TOKENS // NW
CHUNKS_PER_SIDE = PER_SIDE // CHUNK
N_CHUNKS = 2 * CHUNKS_PER_SIDE
W_PER_ROW = SEQ // PER_SIDE
ROWS_UNROLL = 2


@functools.partial(
    pl.kernel,
    out_type=(
        jax.ShapeDtypeStruct((SIDE_TOKENS, EMBED_DIM), jnp.float32),
        jax.ShapeDtypeStruct((SIDE_TOKENS, EMBED_DIM), jnp.float32),
    ),
    mesh=plsc.VectorSubcoreMesh(core_axis_name="c", subcore_axis_name="s"),
    scratch_types=[
        pltpu.VMEM((N_CHUNKS * CHUNK,), jnp.int32),
        pltpu.VMEM((N_CHUNKS * CHUNK, EMBED_DIM), jnp.float32),
        pltpu.VMEM((2, L), jnp.float32),
        pltpu.SemaphoreType.DMA,
        pltpu.SemaphoreType.DMA,
        pltpu.SemaphoreType.DMA,
        pltpu.SemaphoreType.DMA,
        pltpu.SemaphoreType.DMA,
        pltpu.SemaphoreType.DMA,
        pltpu.SemaphoreType.DMA,
        pltpu.SemaphoreType.DMA,
    ],
)
def _embed_kernel(enc_ids, dec_ids, scales, table,
                  enc_out, dec_out,
                  idx_v, rows_v, scale_v, g0, g1, g2, g3, osem, i0, i1, ssem):
    wid = lax.axis_index("s") * NC + lax.axis_index("c")
    b = wid // W_PER_ROW
    col = (wid % W_PER_ROW) * PER_SIDE

    gsems = [g0, g1, g2, g3]
    isems = [i0, i1]
    id_copies = [
        pltpu.async_copy(ids.at[b, pl.ds(col, PER_SIDE)],
                         idx_v.at[pl.ds(side * PER_SIDE, PER_SIDE)], isems[side])
        for side, ids in enumerate((enc_ids, dec_ids))
    ]
    s_copy = pltpu.async_copy(scales, scale_v, ssem)
    gathers = []
    for side in range(2):
        id_copies[side].wait()
        for jj in range(CHUNKS_PER_SIDE):
            j = side * CHUNKS_PER_SIDE + jj
            gathers.append(
                pltpu.async_copy(table.at[idx_v.at[pl.ds(j * CHUNK, CHUNK)]],
                                 rows_v.at[pl.ds(j * CHUNK, CHUNK)], gsems[j])
            )
    s_copy.wait()

    writes = []
    for j in range(N_CHUNKS):
        gathers[j].wait()
        side = j // CHUNKS_PER_SIDE
        s = scale_v[side, :]
        base = j * CHUNK

        def scale_rows(i, carry, base=base, s=s):
            row = base + i * ROWS_UNROLL
            for r in range(ROWS_UNROLL):
                for cs in range(EMBED_DIM // L):
                    sl = pl.ds(cs * L, L)
                    rows_v[row + r, sl] = rows_v[row + r, sl] * s
            return carry

        lax.fori_loop(0, CHUNK // ROWS_UNROLL, scale_rows, 0)

        if j % CHUNKS_PER_SIDE == CHUNKS_PER_SIDE - 1:
            dst = enc_out if side == 0 else dec_out
            writes.append(
                pltpu.async_copy(rows_v.at[pl.ds(side * PER_SIDE, PER_SIDE)],
                                 dst.at[pl.ds(wid * PER_SIDE, PER_SIDE)], osem)
            )
    for w in writes:
        w.wait()


def kernel(input_ids, encoder_embed_scale, decoder_input_ids, decoder_embed_scale, table):
    batch, enc_len = input_ids.shape
    dec_len = decoder_input_ids.shape[1]
    scales = jnp.broadcast_to(
        jnp.stack([encoder_embed_scale, decoder_embed_scale]).astype(jnp.float32)[:, None],
        (2, L),
    )
    enc, dec = _embed_kernel(input_ids.astype(jnp.int32),
                             decoder_input_ids.astype(jnp.int32),
                             scales, table)
    return (enc.reshape(batch, enc_len, EMBED_DIM),
            dec.reshape(batch, dec_len, EMBED_DIM))
